# baseline (speedup 1.0000x reference)
import functools

import jax
import jax.numpy as jnp
from jax import lax
from jax.experimental import pallas as pl
from jax.experimental.pallas import tpu as pltpu
from jax.experimental.pallas import tpu_sc as plsc

_NUM_AUGS = 1000
_NUM_CLS = 100000
_EMBED = 64
_HID = 256
_B = 16384

_NC = 2
_NS = 16
_NW = _NC * _NS

_NCHUNK = 8
_CB = _B // _NCHUNK
_BPW = _CB // _NW

_BS = 1024


def _gather_cls_body(chunk, cls_hbm, idx_hbm, oc, idx_v, rows_v, sem):
    wid = lax.axis_index("s") * _NC + lax.axis_index("c")
    base = wid * _BPW
    pltpu.sync_copy(idx_hbm.at[pl.ds(chunk * _CB + base, _BPW)], idx_v)
    pltpu.async_copy(cls_hbm.at[idx_v], rows_v, sem).wait()
    pltpu.sync_copy(rows_v, oc.at[pl.ds(base, _BPW)])


def _mlp_body(xb, gc, A, W0, b0, W1, b1, W2, b2, Wout, o_ref):
    f32 = jnp.float32
    x = xb[...]
    iot = lax.broadcasted_iota(jnp.int32, (_BS, _NUM_AUGS), 1)
    cnt = ((iot == x[:, 0:1]).astype(f32)
           + (iot == x[:, 1:2]).astype(f32)
           + (iot == x[:, 2:3]).astype(f32))
    aug = lax.dot_general(cnt, A[...], (((1,), (0,)), ((), ())),
                          preferred_element_type=f32)
    h = jnp.concatenate([aug, gc[:, :_EMBED]], axis=1)
    h = lax.dot_general(h, W0[...], (((1,), (1,)), ((), ())),
                        preferred_element_type=f32)
    h = jnp.maximum(h + b0[...], 0.0)
    h = lax.dot_general(h, W1[...], (((1,), (1,)), ((), ())),
                        preferred_element_type=f32)
    h = jnp.maximum(h + b1[...], 0.0)
    h = lax.dot_general(h, W2[...], (((1,), (1,)), ((), ())),
                        preferred_element_type=f32)
    h = jnp.maximum(h + b2[...], 0.0)
    o_ref[...] = jnp.sum(h * Wout[...], axis=1, keepdims=True)


def kernel(x, aug_table, cls_table, W0, b0, W1, b1, W2, b2, Wout, bout):
    aug_z = aug_table.at[_NUM_AUGS - 1].set(0.0)
    idx_cls = x[:, 3]

    mesh = plsc.VectorSubcoreMesh(core_axis_name="c", subcore_axis_name="s")
    emb = jax.ShapeDtypeStruct((_CB, 2 * _EMBED), jnp.float32)
    scratch = [
        pltpu.VMEM((_BPW,), jnp.int32),
        pltpu.VMEM((_BPW, 2 * _EMBED), jnp.float32),
        pltpu.SemaphoreType.DMA,
    ]

    nblk = _CB // _BS
    xspec = pl.BlockSpec((_BS, 4), lambda i: (i, 0))
    gspec = pl.BlockSpec((_BS, 2 * _EMBED), lambda i: (i, 0))
    wspec = lambda r, c: pl.BlockSpec((r, c), lambda i: (0, 0))
    mlp = pl.pallas_call(
        _mlp_body,
        grid=(nblk,),
        in_specs=[
            xspec, gspec,
            wspec(_NUM_AUGS, _EMBED),
            wspec(_HID, 2 * _EMBED),
            wspec(1, _HID),
            wspec(_HID, _HID),
            wspec(1, _HID),
            wspec(_HID, _HID),
            wspec(1, _HID),
            wspec(1, _HID),
        ],
        out_specs=pl.BlockSpec((_BS, 1), lambda i: (i, 0)),
        out_shape=jax.ShapeDtypeStruct((_CB, 1), jnp.float32),
    )

    b0r = b0.reshape(1, _HID)
    b1r = b1.reshape(1, _HID)
    b2r = b2.reshape(1, _HID)

    cls_e = jnp.pad(cls_table, ((0, 0), (0, _EMBED)))

    ys = []
    for c in range(_NCHUNK):
        gather_cls = pl.kernel(
            functools.partial(_gather_cls_body, c),
            mesh=mesh,
            out_type=emb,
            scratch_types=scratch,
        )
        gc = gather_cls(cls_e, idx_cls)
        xc = lax.slice_in_dim(x, c * _CB, (c + 1) * _CB, axis=0)
        ys.append(mlp(xc, gc, aug_z,
                      W0, b0r, W1, b1r, W2, b2r, Wout))
    return jnp.concatenate(ys, axis=0) + bout


# speedup vs baseline: 1.1492x; 1.1492x over previous
import functools

import jax
import jax.numpy as jnp
from jax import lax
from jax.experimental import pallas as pl
from jax.experimental.pallas import tpu as pltpu
from jax.experimental.pallas import tpu_sc as plsc

_NUM_AUGS = 1000
_NUM_CLS = 100000
_EMBED = 64
_HID = 256
_B = 16384

_NC = 2
_NS = 16
_NW = _NC * _NS

_NCHUNK = 4
_CB = _B // _NCHUNK
_BPW = _CB // _NW

_BS = 1024


def _gather_cls_body(chunk, cls_hbm, idx_hbm, oc, idx_v, rows_v, sem):
    wid = lax.axis_index("s") * _NC + lax.axis_index("c")
    base = wid * _BPW
    pltpu.sync_copy(idx_hbm.at[pl.ds(chunk * _CB + base, _BPW)], idx_v)
    pltpu.async_copy(cls_hbm.at[idx_v], rows_v, sem).wait()
    pltpu.sync_copy(rows_v, oc.at[pl.ds(base, _BPW)])


def _mlp_body(xb, gc, A, W0, b0, W1, b1, W2, b2, Wout, o_ref):
    f32 = jnp.float32
    x = xb[...]
    iot = lax.broadcasted_iota(jnp.int32, (_BS, _NUM_AUGS), 1)
    cnt = ((iot == x[:, 0:1]).astype(f32)
           + (iot == x[:, 1:2]).astype(f32)
           + (iot == x[:, 2:3]).astype(f32))
    aug = lax.dot_general(cnt, A[...], (((1,), (0,)), ((), ())),
                          preferred_element_type=f32)
    h = jnp.concatenate([aug, gc[:, :_EMBED]], axis=1)
    h = lax.dot_general(h, W0[...], (((1,), (1,)), ((), ())),
                        preferred_element_type=f32)
    h = jnp.maximum(h + b0[...], 0.0)
    h = lax.dot_general(h, W1[...], (((1,), (1,)), ((), ())),
                        preferred_element_type=f32)
    h = jnp.maximum(h + b1[...], 0.0)
    h = lax.dot_general(h, W2[...], (((1,), (1,)), ((), ())),
                        preferred_element_type=f32)
    h = jnp.maximum(h + b2[...], 0.0)
    o_ref[...] = jnp.sum(h * Wout[...], axis=1, keepdims=True)


def kernel(x, aug_table, cls_table, W0, b0, W1, b1, W2, b2, Wout, bout):
    aug_z = aug_table.at[_NUM_AUGS - 1].set(0.0)
    idx_cls = x[:, 3]

    mesh = plsc.VectorSubcoreMesh(core_axis_name="c", subcore_axis_name="s")
    emb = jax.ShapeDtypeStruct((_CB, 2 * _EMBED), jnp.float32)
    scratch = [
        pltpu.VMEM((_BPW,), jnp.int32),
        pltpu.VMEM((_BPW, 2 * _EMBED), jnp.float32),
        pltpu.SemaphoreType.DMA,
    ]

    nblk = _CB // _BS
    xspec = pl.BlockSpec((_BS, 4), lambda i: (i, 0))
    gspec = pl.BlockSpec((_BS, 2 * _EMBED), lambda i: (i, 0))
    wspec = lambda r, c: pl.BlockSpec((r, c), lambda i: (0, 0))
    mlp = pl.pallas_call(
        _mlp_body,
        grid=(nblk,),
        in_specs=[
            xspec, gspec,
            wspec(_NUM_AUGS, _EMBED),
            wspec(_HID, 2 * _EMBED),
            wspec(1, _HID),
            wspec(_HID, _HID),
            wspec(1, _HID),
            wspec(_HID, _HID),
            wspec(1, _HID),
            wspec(1, _HID),
        ],
        out_specs=pl.BlockSpec((_BS, 1), lambda i: (i, 0)),
        out_shape=jax.ShapeDtypeStruct((_CB, 1), jnp.float32),
    )

    b0r = b0.reshape(1, _HID)
    b1r = b1.reshape(1, _HID)
    b2r = b2.reshape(1, _HID)

    cls_e = jnp.pad(cls_table, ((0, 0), (0, _EMBED)))

    ys = []
    for c in range(_NCHUNK):
        gather_cls = pl.kernel(
            functools.partial(_gather_cls_body, c),
            mesh=mesh,
            out_type=emb,
            scratch_types=scratch,
        )
        gc = gather_cls(cls_e, idx_cls)
        xc = lax.slice_in_dim(x, c * _CB, (c + 1) * _CB, axis=0)
        ys.append(mlp(xc, gc, aug_z,
                      W0, b0r, W1, b1r, W2, b2r, Wout))
    return jnp.concatenate(ys, axis=0) + bout
